# baseline (speedup 1.0000x reference)
import functools

import jax
import jax.numpy as jnp
from jax import lax
from jax.experimental import pallas as pl
from jax.experimental.pallas import tpu as pltpu
from jax.experimental.pallas import tpu_sc as plsc

_LANE = 128
_K = 8
_NBUF = 4

_SC_PARAMS = pltpu.CompilerParams(needs_layout_passes=False)


def _zero_acc(acc_v, n_pad):
    z = jnp.zeros((16,), jnp.float32)

    def zbody(i, c):
        for u in range(8):
            acc_v[pl.ds((i * 8 + u) * 16, 16)] = z
        return c

    lax.fori_loop(0, n_pad // 128, zbody, 0)


def _make_edge_pass(n_pad, real_rows, rows, nc, ns, with_gather):
    nw = nc * ns
    cpt = rows // (nw * _K)
    assert cpt % _NBUF == 0 and cpt > _NBUF
    seg = n_pad // ns
    mesh = plsc.VectorSubcoreMesh(core_axis_name="c", subcore_axis_name="s")

    scratch = [
        pltpu.VMEM((_NBUF, _K * _LANE), jnp.int32),
        pltpu.VMEM((n_pad,), jnp.float32),
    ] + [pltpu.SemaphoreType.DMA] * _NBUF
    if with_gather:
        scratch += [
            pltpu.VMEM((_NBUF, _K * _LANE), jnp.int32),
            pltpu.VMEM((_NBUF, _K * _LANE), jnp.float32),
            pltpu.VMEM_SHARED((n_pad,), jnp.float32),
        ] + [pltpu.SemaphoreType.DMA] * _NBUF

    @functools.partial(
        pl.kernel,
        mesh=mesh,
        out_type=jax.ShapeDtypeStruct((nw, n_pad), jnp.float32),
        scratch_types=scratch,
        compiler_params=_SC_PARAMS,
    )
    def k(*args):
        if with_gather:
            (eidx_hbm, pad_hbm, table_hbm, out_hbm, dst_v, acc_v,
             *rest) = args
            isems = rest[:_NBUF]
            src_v, val_v = rest[_NBUF], rest[_NBUF + 1]
            table_sh = rest[_NBUF + 2]
            gsems = rest[_NBUF + 3:]
        else:
            eidx_hbm, pad_hbm, out_hbm, dst_v, acc_v, *isems = args

        cid = lax.axis_index("c")
        sid = lax.axis_index("s")
        wid = cid * ns + sid
        c_base = wid * cpt
        ones = jnp.ones((16,), jnp.float32)

        if with_gather:
            pltpu.sync_copy(table_hbm.at[pl.ds(sid * seg, seg)],
                            table_sh.at[pl.ds(sid * seg, seg)])
        _zero_acc(acc_v, n_pad)
        if with_gather:
            plsc.subcore_barrier()

        ew = real_rows * _LANE
        cw = _K * _LANE

        def load_idx(c, b):
            r = c * _K

            @pl.when(r < real_rows)
            def _():
                pltpu.async_copy(eidx_hbm.at[pl.ds(ew + r * _LANE, cw)],
                                 dst_v.at[b], isems[b])
                if with_gather:
                    pltpu.async_copy(eidx_hbm.at[pl.ds(r * _LANE, cw)],
                                     src_v.at[b], isems[b])

            @pl.when(r >= real_rows)
            def _():
                pltpu.async_copy(
                    pad_hbm.at[pl.ds((r - real_rows) * _LANE, cw)],
                    dst_v.at[b], isems[b])
                if with_gather:
                    pltpu.async_copy(
                        pad_hbm.at[pl.ds((r - real_rows) * _LANE, cw)],
                        src_v.at[b], isems[b])

        def wait_idx(b):
            pltpu.make_async_copy(eidx_hbm.at[pl.ds(0, cw)], dst_v.at[b],
                                  isems[b]).wait()
            if with_gather:
                pltpu.make_async_copy(eidx_hbm.at[pl.ds(0, cw)],
                                      src_v.at[b], isems[b]).wait()

        def fire_gathers(b):
            for j in range(_K):
                pltpu.async_copy(table_sh.at[src_v.at[b, pl.ds(j * _LANE, _LANE)]],
                                 val_v.at[b, pl.ds(j * _LANE, _LANE)],
                                 gsems[b])

        def drain_gathers(b):
            for j in range(_K):
                pltpu.make_async_copy(
                    table_sh.at[src_v.at[b, pl.ds(j * _LANE, _LANE)]],
                    val_v.at[b, pl.ds(j * _LANE, _LANE)], gsems[b]).wait()

        def consume(b):
            for j in range(_K):
                for u in range(8):
                    idx = dst_v[b, pl.ds(j * _LANE + u * 16, 16)]
                    if with_gather:
                        val = val_v[b, pl.ds(j * _LANE + u * 16, 16)]
                    else:
                        val = ones
                    plsc.addupdate_scatter(acc_v, [idx], val)

        for b in range(_NBUF):
            load_idx(c_base + b, b)

        def body(q, carry):
            c0 = c_base + q * _NBUF
            if with_gather:
                for b in range(_NBUF):
                    wait_idx(b)
                    fire_gathers(b)
                for b in range(_NBUF):
                    drain_gathers(b)
                    consume(b)

                    @pl.when(q < cpt // _NBUF - 1)
                    def _(b=b):
                        load_idx(c0 + b + _NBUF, b)
            else:
                for b in range(_NBUF):
                    wait_idx(b)
                    consume(b)

                    @pl.when(q < cpt // _NBUF - 1)
                    def _(b=b):
                        load_idx(c0 + b + _NBUF, b)
            return carry

        lax.fori_loop(0, cpt // _NBUF, body, 0)
        pltpu.sync_copy(acc_v, out_hbm.at[wid])

    return k


def _node_pass1(degp, x2d):
    nw, r, l = degp.shape

    def body(degp_ref, x_ref, dinv_ref, dinvx_ref):
        deg = degp_ref[0]
        for c in range(1, nw):
            deg = deg + degp_ref[c]
        deg = deg + 1.0
        dinv = lax.rsqrt(deg)
        dinv_ref[...] = dinv
        dinvx_ref[...] = dinv * x_ref[...]

    return pl.pallas_call(
        body,
        out_shape=[jax.ShapeDtypeStruct((r, l), jnp.float32),
                   jax.ShapeDtypeStruct((r, l), jnp.float32)],
    )(degp, x2d)


def _node_pass2(accp, dinv2d, x2d, W1, b1, W2):
    nw, r, l = accp.shape
    f = W1.shape[1]

    def body(accp_ref, dinv_ref, x_ref, w1_ref, b1_ref, w2_ref, t_ref, dinvt_ref):
        acc = accp_ref[0]
        for c in range(1, nw):
            acc = acc + accp_ref[c]
        dinv = dinv_ref[...]
        s1 = dinv * (acc + dinv * x_ref[...])
        t = jnp.zeros((r, l), jnp.float32)
        for k in range(f):
            t = t + jnp.maximum(s1 * w1_ref[0, k] + b1_ref[k], 0.0) * w2_ref[k, 0]
        t_ref[...] = t
        dinvt_ref[...] = dinv * t

    return pl.pallas_call(
        body,
        in_specs=[pl.BlockSpec(memory_space=pltpu.VMEM)] * 3
        + [pl.BlockSpec(memory_space=pltpu.SMEM)] * 3,
        out_shape=[jax.ShapeDtypeStruct((r, l), jnp.float32),
                   jax.ShapeDtypeStruct((r, l), jnp.float32)],
    )(accp, dinv2d, x2d, W1, b1, W2)


def _node_pass3(acc2p, dinv2d, t2d, b2):
    nw, r, l = acc2p.shape

    def body(accp_ref, dinv_ref, t_ref, b2_ref, out_ref):
        acc = accp_ref[0]
        for c in range(1, nw):
            acc = acc + accp_ref[c]
        dinv = dinv_ref[...]
        out_ref[...] = dinv * (acc + dinv * t_ref[...]) + b2_ref[0]

    return pl.pallas_call(
        body,
        in_specs=[pl.BlockSpec(memory_space=pltpu.VMEM)] * 3
        + [pl.BlockSpec(memory_space=pltpu.SMEM)],
        out_shape=jax.ShapeDtypeStruct((r, l), jnp.float32),
    )(acc2p, dinv2d, t2d, b2)


def kernel(x, edge_index, W1, b1, W2, b2):
    n = x.shape[0]
    e = edge_index.shape[1]
    assert e % _LANE == 0
    info = plsc.get_sparse_core_info()
    nc, ns = info.num_cores, info.num_subcores
    nw = nc * ns

    n_pad = ((n + 256 + _LANE * ns - 1) // (_LANE * ns)) * (_LANE * ns)
    spread = n_pad - n
    nr = n_pad // _LANE

    real_rows = e // _LANE
    unit = _NBUF * _K * nw
    rows = -(-real_rows // unit) * unit
    pad_rows = rows - real_rows

    eidx1d = edge_index.reshape(2 * e)
    pad1d = n + (jnp.arange(pad_rows * _LANE, dtype=jnp.int32) % spread)

    xf = jnp.concatenate([x[:, 0], jnp.zeros((n_pad - n,), jnp.float32)])
    x2d = xf.reshape(nr, _LANE)

    deg_pass = _make_edge_pass(n_pad, real_rows, rows, nc, ns, False)
    gs_pass = _make_edge_pass(n_pad, real_rows, rows, nc, ns, True)

    degp = deg_pass(eidx1d, pad1d).reshape(nw, nr, _LANE)
    dinv2d, dinvx2d = _node_pass1(degp, x2d)

    accp = gs_pass(eidx1d, pad1d, dinvx2d.reshape(n_pad))
    t2d, dinvt2d = _node_pass2(accp.reshape(nw, nr, _LANE), dinv2d, x2d, W1, b1, W2)

    acc2p = gs_pass(eidx1d, pad1d, dinvt2d.reshape(n_pad))
    out2d = _node_pass3(acc2p.reshape(nw, nr, _LANE), dinv2d, t2d, b2)

    return out2d.reshape(n_pad)[:n].reshape(n, 1)


# speedup vs baseline: 1.1273x; 1.1273x over previous
import functools

import jax
import jax.numpy as jnp
from jax import lax
from jax.experimental import pallas as pl
from jax.experimental.pallas import tpu as pltpu
from jax.experimental.pallas import tpu_sc as plsc

_LANE = 128
_K = 8
_NBUF = 4

_SC_PARAMS = pltpu.CompilerParams(needs_layout_passes=False)


def _zero_acc(acc_v, n_pad):
    z = jnp.zeros((16,), jnp.float32)

    def zbody(i, c):
        for u in range(8):
            acc_v[pl.ds((i * 8 + u) * 16, 16)] = z
        return c

    lax.fori_loop(0, n_pad // 128, zbody, 0)


def _make_edge_pass(n_pad, real_rows, rows, nc, ns, with_gather):
    nw = nc * ns
    cpt = rows // (nw * _K)
    assert cpt % _NBUF == 0 and cpt > _NBUF
    seg = n_pad // ns
    mesh = plsc.VectorSubcoreMesh(core_axis_name="c", subcore_axis_name="s")

    scratch = [
        pltpu.VMEM((_NBUF, _K, _LANE), jnp.int32),
        pltpu.VMEM((n_pad,), jnp.float32),
    ] + [pltpu.SemaphoreType.DMA] * _NBUF
    if with_gather:
        scratch += [
            pltpu.VMEM((_NBUF, _K, _LANE), jnp.int32),
            pltpu.VMEM((_NBUF, _K, _LANE), jnp.float32),
            pltpu.VMEM_SHARED((n_pad,), jnp.float32),
        ] + [pltpu.SemaphoreType.DMA] * _NBUF

    @functools.partial(
        pl.kernel,
        mesh=mesh,
        out_type=jax.ShapeDtypeStruct((nw, n_pad), jnp.float32),
        scratch_types=scratch,
        compiler_params=_SC_PARAMS,
    )
    def k(*args):
        if with_gather:
            (eidx_hbm, pad_hbm, table_hbm, out_hbm, dst_v, acc_v,
             *rest) = args
            isems = rest[:_NBUF]
            src_v, val_v = rest[_NBUF], rest[_NBUF + 1]
            table_sh = rest[_NBUF + 2]
            gsems = rest[_NBUF + 3:]
        else:
            eidx_hbm, pad_hbm, out_hbm, dst_v, acc_v, *isems = args

        cid = lax.axis_index("c")
        sid = lax.axis_index("s")
        wid = cid * ns + sid
        c_base = wid * cpt
        ones = jnp.ones((16,), jnp.float32)

        if with_gather:
            pltpu.sync_copy(table_hbm.at[pl.ds(sid * seg, seg)],
                            table_sh.at[pl.ds(sid * seg, seg)])
        _zero_acc(acc_v, n_pad)
        if with_gather:
            plsc.subcore_barrier()

        def load_idx(c, b):
            r = c * _K

            @pl.when(r < real_rows)
            def _():
                pltpu.async_copy(eidx_hbm.at[pl.ds(r, _K), 1], dst_v.at[b],
                                 isems[b])
                if with_gather:
                    pltpu.async_copy(eidx_hbm.at[pl.ds(r, _K), 0],
                                     src_v.at[b], isems[b])

            @pl.when(r >= real_rows)
            def _():
                pltpu.async_copy(pad_hbm.at[pl.ds(r - real_rows, _K)],
                                 dst_v.at[b], isems[b])
                if with_gather:
                    pltpu.async_copy(pad_hbm.at[pl.ds(r - real_rows, _K)],
                                     src_v.at[b], isems[b])

        def wait_idx(b):
            pltpu.make_async_copy(eidx_hbm.at[pl.ds(0, _K), 1], dst_v.at[b],
                                  isems[b]).wait()
            if with_gather:
                pltpu.make_async_copy(eidx_hbm.at[pl.ds(0, _K), 0],
                                      src_v.at[b], isems[b]).wait()

        def fire_gathers(b):
            for j in range(_K):
                pltpu.async_copy(table_sh.at[src_v.at[b, j]], val_v.at[b, j],
                                 gsems[b])

        def drain_gathers(b):
            for j in range(_K):
                pltpu.make_async_copy(table_sh.at[src_v.at[b, j]],
                                      val_v.at[b, j], gsems[b]).wait()

        def consume(b):
            for j in range(_K):
                for u in range(8):
                    idx = dst_v[b, j, pl.ds(u * 16, 16)]
                    if with_gather:
                        val = val_v[b, j, pl.ds(u * 16, 16)]
                    else:
                        val = ones
                    plsc.addupdate_scatter(acc_v, [idx], val)

        for b in range(_NBUF):
            load_idx(c_base + b, b)

        def body(q, carry):
            c0 = c_base + q * _NBUF
            if with_gather:
                for b in range(_NBUF):
                    wait_idx(b)
                    fire_gathers(b)
                for b in range(_NBUF):
                    drain_gathers(b)
                    consume(b)

                    @pl.when(q < cpt // _NBUF - 1)
                    def _(b=b):
                        load_idx(c0 + b + _NBUF, b)
            else:
                for b in range(_NBUF):
                    wait_idx(b)
                    consume(b)

                    @pl.when(q < cpt // _NBUF - 1)
                    def _(b=b):
                        load_idx(c0 + b + _NBUF, b)
            return carry

        lax.fori_loop(0, cpt // _NBUF, body, 0)
        pltpu.sync_copy(acc_v, out_hbm.at[wid])

    return k


def _node_pass1(degp, x2d):
    nw, r, l = degp.shape

    def body(degp_ref, x_ref, dinv_ref, dinvx_ref):
        deg = degp_ref[0]
        for c in range(1, nw):
            deg = deg + degp_ref[c]
        deg = deg + 1.0
        dinv = lax.rsqrt(deg)
        dinv_ref[...] = dinv
        dinvx_ref[...] = dinv * x_ref[...]

    return pl.pallas_call(
        body,
        out_shape=[jax.ShapeDtypeStruct((r, l), jnp.float32),
                   jax.ShapeDtypeStruct((r, l), jnp.float32)],
    )(degp, x2d)


def _node_pass2(accp, dinv2d, x2d, W1, b1, W2):
    nw, r, l = accp.shape
    f = W1.shape[1]

    def body(accp_ref, dinv_ref, x_ref, w1_ref, b1_ref, w2_ref, t_ref, dinvt_ref):
        acc = accp_ref[0]
        for c in range(1, nw):
            acc = acc + accp_ref[c]
        dinv = dinv_ref[...]
        s1 = dinv * (acc + dinv * x_ref[...])
        t = jnp.zeros((r, l), jnp.float32)
        for k in range(f):
            t = t + jnp.maximum(s1 * w1_ref[0, k] + b1_ref[k], 0.0) * w2_ref[k, 0]
        t_ref[...] = t
        dinvt_ref[...] = dinv * t

    return pl.pallas_call(
        body,
        in_specs=[pl.BlockSpec(memory_space=pltpu.VMEM)] * 3
        + [pl.BlockSpec(memory_space=pltpu.SMEM)] * 3,
        out_shape=[jax.ShapeDtypeStruct((r, l), jnp.float32),
                   jax.ShapeDtypeStruct((r, l), jnp.float32)],
    )(accp, dinv2d, x2d, W1, b1, W2)


def _node_pass3(acc2p, dinv2d, t2d, b2):
    nw, r, l = acc2p.shape

    def body(accp_ref, dinv_ref, t_ref, b2_ref, out_ref):
        acc = accp_ref[0]
        for c in range(1, nw):
            acc = acc + accp_ref[c]
        dinv = dinv_ref[...]
        out_ref[...] = dinv * (acc + dinv * t_ref[...]) + b2_ref[0]

    return pl.pallas_call(
        body,
        in_specs=[pl.BlockSpec(memory_space=pltpu.VMEM)] * 3
        + [pl.BlockSpec(memory_space=pltpu.SMEM)],
        out_shape=jax.ShapeDtypeStruct((r, l), jnp.float32),
    )(acc2p, dinv2d, t2d, b2)


def kernel(x, edge_index, W1, b1, W2, b2):
    n = x.shape[0]
    e = edge_index.shape[1]
    assert e % _LANE == 0
    info = plsc.get_sparse_core_info()
    nc, ns = info.num_cores, info.num_subcores
    nw = nc * ns

    n_pad = ((n + 256 + _LANE * ns - 1) // (_LANE * ns)) * (_LANE * ns)
    spread = n_pad - n
    nr = n_pad // _LANE

    real_rows = e // _LANE
    unit = _NBUF * _K * nw
    rows = -(-real_rows // unit) * unit
    pad_rows = rows - real_rows

    eidx3d = edge_index.reshape(2, real_rows, _LANE).transpose(1, 0, 2)
    pad2d = (n + (jnp.arange(pad_rows * _LANE, dtype=jnp.int32) % spread)
             ).reshape(pad_rows, _LANE)

    xf = jnp.concatenate([x[:, 0], jnp.zeros((n_pad - n,), jnp.float32)])
    x2d = xf.reshape(nr, _LANE)

    deg_pass = _make_edge_pass(n_pad, real_rows, rows, nc, ns, False)
    gs_pass = _make_edge_pass(n_pad, real_rows, rows, nc, ns, True)

    degp = deg_pass(eidx3d, pad2d).reshape(nw, nr, _LANE)
    dinv2d, dinvx2d = _node_pass1(degp, x2d)

    accp = gs_pass(eidx3d, pad2d, dinvx2d.reshape(n_pad))
    t2d, dinvt2d = _node_pass2(accp.reshape(nw, nr, _LANE), dinv2d, x2d, W1, b1, W2)

    acc2p = gs_pass(eidx3d, pad2d, dinvt2d.reshape(n_pad))
    out2d = _node_pass3(acc2p.reshape(nw, nr, _LANE), dinv2d, t2d, b2)

    return out2d.reshape(n_pad)[:n].reshape(n, 1)
